# baseline (speedup 1.0000x reference)
import functools

import jax
import jax.numpy as jnp
from jax.experimental import pallas as pl
from jax.experimental.pallas import tpu as pltpu

E = 8
TOPK = 2
T = 256
EBT = T
NP = E * EBT
P2 = TOPK * T


def _routing(logits):
    m = jnp.max(logits, axis=1, keepdims=True)
    p = jnp.exp(logits - m)
    p = p / jnp.sum(p, axis=1, keepdims=True)

    eidx = jax.lax.broadcasted_iota(jnp.int32, (T, E), 1)
    m1 = jnp.max(p, axis=1, keepdims=True)
    a1 = jnp.min(jnp.where(p == m1, eidx, E), axis=1, keepdims=True)
    p2 = jnp.where(eidx == a1, -1.0, p)
    m2 = jnp.max(p2, axis=1, keepdims=True)
    a2 = jnp.min(jnp.where(p2 == m2, eidx, E), axis=1, keepdims=True)
    s = m1 + m2
    w1 = m1 / s
    w2 = m2 / s

    e_pairs = jnp.concatenate([a1, a2], axis=0)
    w_pairs = jnp.concatenate([w1, w2], axis=0)
    tio = jax.lax.broadcasted_iota(jnp.int32, (T, 1), 0).astype(jnp.float32)
    t_pairs = jnp.concatenate([tio, tio], axis=0)

    oh = (e_pairs == jax.lax.broadcasted_iota(jnp.int32, (P2, E), 1))
    ohf = oh.astype(jnp.float32)

    pr = jax.lax.broadcasted_iota(jnp.int32, (P2, P2), 0)
    pc = jax.lax.broadcasted_iota(jnp.int32, (P2, P2), 1)
    lt = (pc <= pr).astype(jnp.float32)
    incl = jnp.dot(lt, ohf, preferred_element_type=jnp.float32)
    rank = jnp.sum((incl - 1.0) * ohf, axis=1, keepdims=True)
    pos = e_pairs.astype(jnp.float32) * EBT + rank

    slot_iota = jax.lax.broadcasted_iota(jnp.int32, (P2, NP), 1).astype(
        jnp.float32)
    at = (pos == slot_iota).astype(jnp.float32)
    tw = jnp.concatenate([t_pairs, w_pairs], axis=1)
    cdims = (((0,), (0,)), ((), ()))
    idw = jax.lax.dot_general(
        at, tw, cdims, preferred_element_type=jnp.float32)
    return idw[:, 0:1], idw[:, 1:2]


def _moe_kernel(g_ref, x_ref, gu_hbm, dn_hbm, out_ref,
                gu_buf, dn_buf, gu_sem, dn_sem, *, ff):
    def gu_copy(e, slot, half):
        sl = pl.ds(half * ff, ff)
        return pltpu.make_async_copy(gu_hbm.at[e, sl], gu_buf.at[slot, sl],
                                     gu_sem.at[slot, half])

    def dn_copy(e, slot):
        return pltpu.make_async_copy(dn_hbm.at[e], dn_buf.at[slot],
                                     dn_sem.at[slot])

    def start_all(e, slot):
        gu_copy(e, slot, 0).start()
        gu_copy(e, slot, 1).start()
        dn_copy(e, slot).start()

    start_all(0, 0)
    start_all(1, 1)

    ids_all, w_all = _routing(g_ref[...])
    x = x_ref[...]
    tcol = jax.lax.broadcasted_iota(jnp.int32, (EBT, T), 1).astype(
        jnp.float32)
    cdims = (((1,), (1,)), ((), ()))
    sdims = (((0,), (0,)), ((), ()))

    for e in range(E):
        slot = e % 2
        base = e * EBT
        ids = ids_all[base:base + EBT, :]
        w = w_all[base:base + EBT, :]
        perm = (ids == tcol).astype(jnp.float32)
        xg = jnp.dot(perm, x,
                     preferred_element_type=jnp.float32)
        xb = xg.astype(jnp.bfloat16)
        gu_copy(e, slot, 0).wait()
        g = jax.lax.dot_general(
            xb, gu_buf[slot, :ff], cdims,
            preferred_element_type=jnp.float32)
        gu_copy(e, slot, 1).wait()
        u = jax.lax.dot_general(
            xb, gu_buf[slot, ff:], cdims,
            preferred_element_type=jnp.float32)
        h = g * jax.lax.logistic(g) * u
        dn_copy(e, slot).wait()
        dn = dn_buf[slot]
        y = jax.lax.dot_general(
            h.astype(jnp.bfloat16), dn, cdims,
            preferred_element_type=jnp.float32)
        y = y * w
        if e + 2 < E:
            start_all(e + 2, slot)
        contrib = jax.lax.dot_general(
            perm, y, sdims, preferred_element_type=jnp.float32)
        if e == 0:
            out_ref[...] = contrib
        else:
            out_ref[...] += contrib


def kernel(x, gating_output, gate_up_proj, down_proj):
    t, d = x.shape
    ff2 = gate_up_proj.shape[1]
    ff = ff2 // 2

    out = pl.pallas_call(
        functools.partial(_moe_kernel, ff=ff),
        in_specs=[
            pl.BlockSpec(memory_space=pltpu.MemorySpace.VMEM),
            pl.BlockSpec(memory_space=pltpu.MemorySpace.VMEM),
            pl.BlockSpec(memory_space=pltpu.MemorySpace.HBM),
            pl.BlockSpec(memory_space=pltpu.MemorySpace.HBM),
        ],
        out_specs=pl.BlockSpec(memory_space=pltpu.MemorySpace.VMEM),
        scratch_shapes=[
            pltpu.VMEM((2, ff2, d), jnp.float32),
            pltpu.VMEM((2, d, ff), jnp.float32),
            pltpu.SemaphoreType.DMA((2, 2)),
            pltpu.SemaphoreType.DMA((2,)),
        ],
        out_shape=jax.ShapeDtypeStruct((t, d), jnp.float32),
    )(gating_output, x, gate_up_proj, down_proj)
    return out
